# Initial kernel scaffold; baseline (speedup 1.0000x reference)
import jax
import jax.numpy as jnp
from jax.experimental import pallas as pl


def kernel(x, edge_index, W1, b1, W2, b2):
    raise NotImplementedError("write your pallas kernel here")


# speedup vs baseline: 9.3961x; 9.3961x over previous
import functools

import jax
import jax.numpy as jnp
from jax import lax
from jax.experimental import pallas as pl
from jax.experimental.pallas import tpu as pltpu
from jax.experimental.pallas import tpu_sc as plsc

N = 10000
D = 128
E = 320000
NC = 2
NS = 16
NW = NC * NS
G = 128
CH = 80
CHB = 16
EROWS = NW * CH
E_PAD = EROWS * G
NR = 10240
RPT = NR // NS
BR = 1000


def _worker_id():
    return lax.axis_index("s") * NC + lax.axis_index("c")


@functools.lru_cache(maxsize=None)
def _sc_degree():
    mesh = plsc.VectorSubcoreMesh(
        core_axis_name="c", subcore_axis_name="s", num_cores=NC, num_subcores=NS
    )

    def body(dst_hbm, out_hbm, dst_v, ones_v, zero_v, acc_sh, sem):
        c = lax.axis_index("c")
        s = lax.axis_index("s")
        wid = _worker_id()
        e0 = jnp.where(
            lax.iota(jnp.int32, 16) == 0, jnp.float32(1.0), jnp.float32(0.0)
        )
        z16 = jnp.zeros((16,), jnp.float32)

        def initrow(i, carry):
            ones_v[i, :] = e0
            zero_v[i, :] = z16
            return carry

        lax.fori_loop(0, G, initrow, 0)

        def zrow(k, carry):
            pltpu.sync_copy(zero_v, acc_sh.at[pl.ds(s * RPT + k * G, G)])
            return carry

        lax.fori_loop(0, RPT // G, zrow, 0)
        plsc.subcore_barrier()

        pltpu.sync_copy(dst_hbm.at[pl.ds(wid * CH, CH)], dst_v)

        def addgrp(g, carry):
            pltpu.sync_copy(ones_v, acc_sh.at[dst_v.at[g]], add=True)
            return carry

        lax.fori_loop(0, CH, addgrp, 0)
        plsc.subcore_barrier()
        pltpu.sync_copy(
            acc_sh.at[pl.ds(s * RPT, RPT)], out_hbm.at[c, pl.ds(s * RPT, RPT)]
        )

    return pl.kernel(
        body,
        out_type=jax.ShapeDtypeStruct((NC, NR, 16), jnp.float32),
        mesh=mesh,
        scratch_types=[
            pltpu.VMEM((CH, G), jnp.int32),
            pltpu.VMEM((G, 16), jnp.float32),
            pltpu.VMEM((G, 16), jnp.float32),
            pltpu.VMEM_SHARED((NR, 16), jnp.float32),
            pltpu.SemaphoreType.DMA,
        ],
    )


@functools.lru_cache(maxsize=None)
def _sc_agg():
    mesh = plsc.VectorSubcoreMesh(
        core_axis_name="c", subcore_axis_name="s", num_cores=NC, num_subcores=NS
    )

    def body(h_hbm, src_hbm, dst_hbm, out_hbm, src_v, dst_v, rows_a, rows_b,
             acc_sh, sem_a, sem_b):
        c = lax.axis_index("c")
        s = lax.axis_index("s")
        wid = _worker_id()
        z16 = jnp.zeros((16,), jnp.float32)

        def zbuf(i, carry):
            for j in range(D // 16):
                rows_a[i, pl.ds(j * 16, 16)] = z16
            return carry

        lax.fori_loop(0, G, zbuf, 0)

        def zrow(k, carry):
            pltpu.sync_copy(rows_a, acc_sh.at[pl.ds(s * RPT + k * G, G)])
            return carry

        lax.fori_loop(0, RPT // G, zrow, 0)
        plsc.subcore_barrier()

        def blk(t, carry):
            pltpu.sync_copy(src_hbm.at[pl.ds(wid * CH + t * CHB, CHB)], src_v)
            pltpu.sync_copy(dst_hbm.at[pl.ds(wid * CH + t * CHB, CHB)], dst_v)

            def pair(p, c2):
                g = 2 * p
                cp_a = pltpu.async_copy(h_hbm.at[src_v.at[g]], rows_a, sem_a)
                cp_b = pltpu.async_copy(h_hbm.at[src_v.at[g + 1]], rows_b, sem_b)
                cp_a.wait()
                pltpu.sync_copy(rows_a, acc_sh.at[dst_v.at[g]], add=True)
                cp_b.wait()
                pltpu.sync_copy(rows_b, acc_sh.at[dst_v.at[g + 1]], add=True)
                return c2

            lax.fori_loop(0, CHB // 2, pair, 0)
            return carry

        lax.fori_loop(0, CH // CHB, blk, 0)
        plsc.subcore_barrier()
        pltpu.sync_copy(
            acc_sh.at[pl.ds(s * RPT, RPT)], out_hbm.at[c, pl.ds(s * RPT, RPT)]
        )

    return pl.kernel(
        body,
        out_type=jax.ShapeDtypeStruct((NC, NR, D), jnp.float32),
        mesh=mesh,
        scratch_types=[
            pltpu.VMEM((CHB, G), jnp.int32),
            pltpu.VMEM((CHB, G), jnp.int32),
            pltpu.VMEM((G, D), jnp.float32),
            pltpu.VMEM((G, D), jnp.float32),
            pltpu.VMEM_SHARED((NR, D), jnp.float32),
            pltpu.SemaphoreType.DMA,
            pltpu.SemaphoreType.DMA,
        ],
    )


def _dinv(d0_ref, d1_ref):
    deg = d0_ref[:, 0:1] + d1_ref[:, 0:1] + jnp.float32(1.0)
    return lax.rsqrt(deg)


def _tc_in_body(x_ref, w_ref, d0_ref, d1_ref, o_ref):
    h = jnp.dot(x_ref[...], w_ref[...], preferred_element_type=jnp.float32)
    o_ref[...] = h * _dinv(d0_ref, d1_ref)


def _tc_mid_body(h_ref, p0_ref, p1_ref, d0_ref, d1_ref, w_ref, b_ref, o_ref):
    dinv = _dinv(d0_ref, d1_ref)
    a = (h_ref[...] + p0_ref[...] + p1_ref[...]) * dinv + b_ref[...]
    r = jnp.maximum(a, jnp.float32(0.0))
    o_ref[...] = jnp.dot(r, w_ref[...], preferred_element_type=jnp.float32) * dinv


def _tc_out_body(h_ref, p0_ref, p1_ref, d0_ref, d1_ref, b_ref, o_ref):
    dinv = _dinv(d0_ref, d1_ref)
    o_ref[...] = (h_ref[...] + p0_ref[...] + p1_ref[...]) * dinv + b_ref[...]


_ROWS = pl.BlockSpec((BR, D), lambda i: (i, 0))
_DEGB = pl.BlockSpec((BR, 16), lambda i: (i, 0))
_WFULL = pl.BlockSpec((D, D), lambda i: (0, 0))
_BFULL = pl.BlockSpec((1, D), lambda i: (0, 0))
_OUT = jax.ShapeDtypeStruct((N, D), jnp.float32)


def _tc_in(x, w1, d0, d1):
    return pl.pallas_call(
        _tc_in_body,
        grid=(N // BR,),
        in_specs=[_ROWS, _WFULL, _DEGB, _DEGB],
        out_specs=_ROWS,
        out_shape=_OUT,
    )(x, w1, d0, d1)


def _tc_mid(h, p0, p1, d0, d1, w2, b1):
    return pl.pallas_call(
        _tc_mid_body,
        grid=(N // BR,),
        in_specs=[_ROWS, _ROWS, _ROWS, _DEGB, _DEGB, _WFULL, _BFULL],
        out_specs=_ROWS,
        out_shape=_OUT,
    )(h, p0, p1, d0, d1, w2, b1)


def _tc_out(h, p0, p1, d0, d1, b2):
    return pl.pallas_call(
        _tc_out_body,
        grid=(N // BR,),
        in_specs=[_ROWS, _ROWS, _ROWS, _DEGB, _DEGB, _BFULL],
        out_specs=_ROWS,
        out_shape=_OUT,
    )(h, p0, p1, d0, d1, b2)


def kernel(x, edge_index, W1, b1, W2, b2):
    src = edge_index[0].astype(jnp.int32)
    dst = edge_index[1].astype(jnp.int32)
    pad = E_PAD - E
    src_p = jnp.concatenate([src, jnp.zeros((pad,), jnp.int32)]).reshape(EROWS, G)
    dst_p = jnp.concatenate([dst, jnp.full((pad,), N, jnp.int32)]).reshape(EROWS, G)

    deg = _sc_degree()(dst_p)
    d0 = deg[0, :N]
    d1 = deg[1, :N]

    h1 = _tc_in(x, W1, d0, d1)
    a1 = _sc_agg()(h1, src_p, dst_p)
    h2 = _tc_mid(h1, a1[0, :N], a1[1, :N], d0, d1, W2, b1.reshape(1, D))
    a2 = _sc_agg()(h2, src_p, dst_p)
    return _tc_out(h2, a2[0, :N], a2[1, :N], d0, d1, b2.reshape(1, D))


# speedup vs baseline: 10.0779x; 1.0726x over previous
import functools

import jax
import jax.numpy as jnp
from jax import lax
from jax.experimental import pallas as pl
from jax.experimental.pallas import tpu as pltpu
from jax.experimental.pallas import tpu_sc as plsc

N = 10000
D = 128
E = 320000
NC = 2
NS = 16
NW = NC * NS
G = 128
CH = 80
CHB = 16
EROWS = NW * CH
E_PAD = EROWS * G
NR = 10240
RPT = NR // NS
BR = 1000


def _worker_id():
    return lax.axis_index("s") * NC + lax.axis_index("c")


@functools.lru_cache(maxsize=None)
def _sc_degree():
    mesh = plsc.VectorSubcoreMesh(
        core_axis_name="c", subcore_axis_name="s", num_cores=NC, num_subcores=NS
    )

    def body(dst_hbm, out_hbm, dst_v, ones_v, zero_v, acc_sh, sem):
        c = lax.axis_index("c")
        s = lax.axis_index("s")
        wid = _worker_id()
        e0 = jnp.where(
            lax.iota(jnp.int32, 16) == 0, jnp.float32(1.0), jnp.float32(0.0)
        )
        z16 = jnp.zeros((16,), jnp.float32)

        def initrow(i, carry):
            ones_v[i, :] = e0
            zero_v[i, :] = z16
            return carry

        lax.fori_loop(0, G, initrow, 0)

        def zrow(k, carry):
            pltpu.sync_copy(zero_v, acc_sh.at[pl.ds(s * RPT + k * G, G)])
            return carry

        lax.fori_loop(0, RPT // G, zrow, 0)
        plsc.subcore_barrier()

        pltpu.sync_copy(dst_hbm.at[pl.ds(wid * CH, CH)], dst_v)

        def addgrp(g, carry):
            pltpu.sync_copy(ones_v, acc_sh.at[dst_v.at[g]], add=True)
            return carry

        lax.fori_loop(0, CH, addgrp, 0)
        plsc.subcore_barrier()
        pltpu.sync_copy(
            acc_sh.at[pl.ds(s * RPT, RPT)], out_hbm.at[c, pl.ds(s * RPT, RPT)]
        )

    return pl.kernel(
        body,
        out_type=jax.ShapeDtypeStruct((NC, NR, 16), jnp.float32),
        mesh=mesh,
        scratch_types=[
            pltpu.VMEM((CH, G), jnp.int32),
            pltpu.VMEM((G, 16), jnp.float32),
            pltpu.VMEM((G, 16), jnp.float32),
            pltpu.VMEM_SHARED((NR, 16), jnp.float32),
            pltpu.SemaphoreType.DMA,
        ],
    )


@functools.lru_cache(maxsize=None)
def _sc_agg():
    mesh = plsc.VectorSubcoreMesh(
        core_axis_name="c", subcore_axis_name="s", num_cores=NC, num_subcores=NS
    )

    def body(h_hbm, src_hbm, dst_hbm, out_hbm, src_v, dst_v, rows_a, rows_b,
             acc_sh, sem_ga, sem_gb):
        c = lax.axis_index("c")
        s = lax.axis_index("s")
        wid = _worker_id()
        z16 = jnp.zeros((16,), jnp.float32)

        def zbuf(i, carry):
            for j in range(D // 16):
                rows_a[i, pl.ds(j * 16, 16)] = z16
            return carry

        lax.fori_loop(0, G, zbuf, 0)

        def zrow(k, carry):
            pltpu.sync_copy(rows_a, acc_sh.at[pl.ds(s * RPT + k * G, G)])
            return carry

        lax.fori_loop(0, RPT // G, zrow, 0)
        plsc.subcore_barrier()

        bufs = (rows_a, rows_b)
        gsems = (sem_ga, sem_gb)

        def blk(t, carry):
            pltpu.sync_copy(src_hbm.at[pl.ds(wid * CH + t * CHB, CHB)], src_v)
            pltpu.sync_copy(dst_hbm.at[pl.ds(wid * CH + t * CHB, CHB)], dst_v)
            gh = [None, None]
            for g in range(CHB):
                b = g % 2
                gh[b] = pltpu.async_copy(
                    h_hbm.at[src_v.at[g]], bufs[b], gsems[b]
                )
                if g >= 1:
                    b1 = (g - 1) % 2
                    gh[b1].wait()
                    pltpu.sync_copy(bufs[b1], acc_sh.at[dst_v.at[g - 1]], add=True)
            blast = (CHB - 1) % 2
            gh[blast].wait()
            pltpu.sync_copy(bufs[blast], acc_sh.at[dst_v.at[CHB - 1]], add=True)
            return carry

        lax.fori_loop(0, CH // CHB, blk, 0)
        plsc.subcore_barrier()
        pltpu.sync_copy(
            acc_sh.at[pl.ds(s * RPT, RPT)], out_hbm.at[c, pl.ds(s * RPT, RPT)]
        )

    return pl.kernel(
        body,
        out_type=jax.ShapeDtypeStruct((NC, NR, D), jnp.float32),
        mesh=mesh,
        scratch_types=[
            pltpu.VMEM((CHB, G), jnp.int32),
            pltpu.VMEM((CHB, G), jnp.int32),
            pltpu.VMEM((G, D), jnp.float32),
            pltpu.VMEM((G, D), jnp.float32),
            pltpu.VMEM_SHARED((NR, D), jnp.float32),
            pltpu.SemaphoreType.DMA,
            pltpu.SemaphoreType.DMA,
        ],
    )


def _dinv(d0_ref, d1_ref):
    deg = d0_ref[:, 0:1] + d1_ref[:, 0:1] + jnp.float32(1.0)
    return lax.rsqrt(deg)


def _tc_in_body(x_ref, w_ref, d0_ref, d1_ref, o_ref):
    h = jnp.dot(x_ref[...], w_ref[...], preferred_element_type=jnp.float32)
    o_ref[...] = h * _dinv(d0_ref, d1_ref)


def _tc_mid_body(h_ref, p0_ref, p1_ref, d0_ref, d1_ref, w_ref, b_ref, o_ref):
    dinv = _dinv(d0_ref, d1_ref)
    a = (h_ref[...] + p0_ref[...] + p1_ref[...]) * dinv + b_ref[...]
    r = jnp.maximum(a, jnp.float32(0.0))
    o_ref[...] = jnp.dot(r, w_ref[...], preferred_element_type=jnp.float32) * dinv


def _tc_out_body(h_ref, p0_ref, p1_ref, d0_ref, d1_ref, b_ref, o_ref):
    dinv = _dinv(d0_ref, d1_ref)
    o_ref[...] = (h_ref[...] + p0_ref[...] + p1_ref[...]) * dinv + b_ref[...]


_ROWS = pl.BlockSpec((BR, D), lambda i: (i, 0))
_DEGB = pl.BlockSpec((BR, 16), lambda i: (i, 0))
_WFULL = pl.BlockSpec((D, D), lambda i: (0, 0))
_BFULL = pl.BlockSpec((1, D), lambda i: (0, 0))
_OUT = jax.ShapeDtypeStruct((N, D), jnp.float32)


def _tc_in(x, w1, d0, d1):
    return pl.pallas_call(
        _tc_in_body,
        grid=(N // BR,),
        in_specs=[_ROWS, _WFULL, _DEGB, _DEGB],
        out_specs=_ROWS,
        out_shape=_OUT,
    )(x, w1, d0, d1)


def _tc_mid(h, p0, p1, d0, d1, w2, b1):
    return pl.pallas_call(
        _tc_mid_body,
        grid=(N // BR,),
        in_specs=[_ROWS, _ROWS, _ROWS, _DEGB, _DEGB, _WFULL, _BFULL],
        out_specs=_ROWS,
        out_shape=_OUT,
    )(h, p0, p1, d0, d1, w2, b1)


def _tc_out(h, p0, p1, d0, d1, b2):
    return pl.pallas_call(
        _tc_out_body,
        grid=(N // BR,),
        in_specs=[_ROWS, _ROWS, _ROWS, _DEGB, _DEGB, _BFULL],
        out_specs=_ROWS,
        out_shape=_OUT,
    )(h, p0, p1, d0, d1, b2)


def kernel(x, edge_index, W1, b1, W2, b2):
    src = edge_index[0].astype(jnp.int32)
    dst = edge_index[1].astype(jnp.int32)
    pad = E_PAD - E
    src_p = jnp.concatenate([src, jnp.zeros((pad,), jnp.int32)]).reshape(EROWS, G)
    dst_p = jnp.concatenate([dst, jnp.full((pad,), N, jnp.int32)]).reshape(EROWS, G)

    deg = _sc_degree()(dst_p)
    d0 = deg[0, :N]
    d1 = deg[1, :N]

    h1 = _tc_in(x, W1, d0, d1)
    a1 = _sc_agg()(h1, src_p, dst_p)
    h2 = _tc_mid(h1, a1[0, :N], a1[1, :N], d0, d1, W2, b1.reshape(1, D))
    a2 = _sc_agg()(h2, src_p, dst_p)
    return _tc_out(h2, a2[0, :N], a2[1, :N], d0, d1, b2.reshape(1, D))
